# Initial kernel scaffold; baseline (speedup 1.0000x reference)
import jax
import jax.numpy as jnp
from jax.experimental import pallas as pl


def kernel(x, values, bias, indices):
    raise NotImplementedError("write your pallas kernel here")


# speedup vs baseline: 27.0024x; 27.0024x over previous
import dataclasses
import functools

import jax
import jax.numpy as jnp
from jax import lax
from jax.experimental import pallas as pl
from jax.experimental.pallas import tpu as pltpu
from jax.experimental.pallas import tpu_sc as plsc

B = 8
N = 65536
M = 65536
E = 4194304

NC_CORES = 2
NS = 16
CH = 2048
CR = CH // 128
ET = E // 4
NCHUNK = ET // CH
HM = M // 2
ZW = 4096


def _sc_kernel(x_hbm, src_hbm, val_hbm, dst_hbm, out_hbm,
               x_v, src_v0, src_v1, val_v0, val_v1, dst_v0, dst_v1,
               msg_v0, msg_v1, zbuf,
               acc_sh, sem_x, sem_in0, sem_in1, sem_sc):
    c = lax.axis_index("c")
    s = lax.axis_index("s")
    b = s // 2
    q = c * 2 + (s % 2)
    half = s % 2
    ebase = q * ET
    rbase = q * (ET // 128)

    xcp = pltpu.async_copy(x_hbm.at[b], x_v, sem_x)

    @pl.loop(0, ZW, step=16)
    def _zero(i):
        zbuf[pl.ds(i, 16)] = jnp.zeros((16,), jnp.float32)

    for k in range(HM // ZW):
        pltpu.sync_copy(zbuf, acc_sh.at[b, pl.ds(half * HM + k * ZW, ZW)])
    plsc.subcore_barrier()

    slots = ((src_v0, val_v0, dst_v0, msg_v0, sem_in0),
             (src_v1, val_v1, dst_v1, msg_v1, sem_in1))

    def start_inputs(g, slot):
        sv, vv, dv, _, sem = slots[slot]
        base = ebase + g * CH
        pltpu.async_copy(src_hbm.at[pl.ds(base, CH)], sv, sem)
        pltpu.async_copy(val_hbm.at[pl.ds(base, CH)], vv, sem)
        pltpu.async_copy(dst_hbm.at[pl.ds(rbase + g * CR, CR)], dv, sem)

    def wait_inputs(slot):
        sv, vv, dv, _, sem = slots[slot]
        pltpu.make_async_copy(src_hbm.at[pl.ds(0, CH)], sv, sem).wait()
        pltpu.make_async_copy(val_hbm.at[pl.ds(0, CH)], vv, sem).wait()
        pltpu.make_async_copy(dst_hbm.at[pl.ds(0, CR)], dv, sem).wait()

    start_inputs(0, 0)
    xcp.wait()

    def process(g, slot):
        wait_inputs(slot)
        sv, vv, dv, mv, _ = slots[slot]

        @pl.loop(0, CH, step=16, unroll=8)
        def _compute(i):
            idx = sv[pl.ds(i, 16)]
            val = vv[pl.ds(i, 16)]
            xg = plsc.load_gather(x_v, [idx])
            mv[pl.ds(i, 16)] = xg * val

        acc_b = acc_sh.at[b]
        cps = []
        for r in range(CR):
            cps.append(pltpu.async_copy(
                mv.at[pl.ds(r * 128, 128)],
                acc_b.at[dv.at[r]],
                sem_sc, add=True))
        for cp in cps:
            cp.wait()

    @pl.loop(0, NCHUNK, step=2)
    def _main(g):
        start_inputs(g + 1, 1)
        process(g, 0)

        @pl.when(g + 2 < NCHUNK)
        def _():
            start_inputs(g + 2, 0)

        process(g + 1, 1)

    plsc.subcore_barrier()
    pltpu.sync_copy(acc_sh.at[b, pl.ds(half * HM, HM)],
                    out_hbm.at[c, b, pl.ds(half * HM, HM)])


def _combine_body(p_ref, b_ref, o_ref):
    o_ref[...] = p_ref[0] + p_ref[1] + b_ref[...]


def kernel(x, values, bias, indices):
    src = indices[0].astype(jnp.int32)
    dst = indices[1].astype(jnp.int32).reshape(E // 128, 128)
    xb = x.reshape(B, N)

    mesh = plsc.VectorSubcoreMesh(core_axis_name="c", subcore_axis_name="s")
    cp = pltpu.CompilerParams(use_tc_tiling_on_sc=False)
    if "needs_layout_passes" in pltpu.CompilerParams.__dataclass_fields__:
        cp = dataclasses.replace(cp, needs_layout_passes=False)
    sc = pl.kernel(
        _sc_kernel,
        out_type=jax.ShapeDtypeStruct((NC_CORES, B, M), jnp.float32),
        mesh=mesh,
        scratch_types=[
            pltpu.VMEM((N,), jnp.float32),
            pltpu.VMEM((CH,), jnp.int32),
            pltpu.VMEM((CH,), jnp.int32),
            pltpu.VMEM((CH,), jnp.float32),
            pltpu.VMEM((CH,), jnp.float32),
            pltpu.VMEM((CR, 128), jnp.int32),
            pltpu.VMEM((CR, 128), jnp.int32),
            pltpu.VMEM((CH,), jnp.float32),
            pltpu.VMEM((CH,), jnp.float32),
            pltpu.VMEM((ZW,), jnp.float32),
            pltpu.VMEM_SHARED((B, M), jnp.float32),
            pltpu.SemaphoreType.DMA,
            pltpu.SemaphoreType.DMA,
            pltpu.SemaphoreType.DMA,
            pltpu.SemaphoreType.DMA,
        ],
        compiler_params=cp,
    )
    partial = sc(xb, src, values, dst)

    bl = 2048
    out = pl.pallas_call(
        _combine_body,
        out_shape=jax.ShapeDtypeStruct((B, M), jnp.float32),
        grid=(M // bl,),
        in_specs=[
            pl.BlockSpec((NC_CORES, B, bl), lambda i: (0, 0, i)),
            pl.BlockSpec((1, bl), lambda i: (0, i)),
        ],
        out_specs=pl.BlockSpec((B, bl), lambda i: (0, i)),
    )(partial, bias.reshape(1, M))
    return out.reshape(B, M, 1)


# speedup vs baseline: 52.6951x; 1.9515x over previous
import dataclasses
import functools

import jax
import jax.numpy as jnp
from jax import lax
from jax.experimental import pallas as pl
from jax.experimental.pallas import tpu as pltpu
from jax.experimental.pallas import tpu_sc as plsc

B = 8
N = 65536
M = 65536
E = 4194304

NC_CORES = 2
NS = 16
CH = 1024
CR = CH // 128
ET = E // 4
NCHUNK = ET // CH
HM = M // 2
ZW = 4096
NSLOT = 4


def _sc_kernel(x_hbm, src_hbm, val_hbm, dst_hbm, out_hbm, x_v, *rest):
    bufs = rest[:4 * NSLOT]
    zbuf = rest[4 * NSLOT]
    acc_sh = rest[4 * NSLOT + 1]
    sem_x = rest[4 * NSLOT + 2]
    in_sems = rest[4 * NSLOT + 3:4 * NSLOT + 3 + NSLOT]
    sc_sems = rest[4 * NSLOT + 3 + NSLOT:4 * NSLOT + 3 + 2 * NSLOT]
    slots = tuple(
        (bufs[4 * k], bufs[4 * k + 1], bufs[4 * k + 2], bufs[4 * k + 3],
         in_sems[k], sc_sems[k])
        for k in range(NSLOT))

    c = lax.axis_index("c")
    s = lax.axis_index("s")
    b = s // 2
    q = c * 2 + (s % 2)
    half = s % 2
    ebase = q * ET
    rbase = q * (ET // 128)

    xcp = pltpu.async_copy(x_hbm.at[b], x_v, sem_x)

    @pl.loop(0, ZW, step=16)
    def _zero(i):
        zbuf[pl.ds(i, 16)] = jnp.zeros((16,), jnp.float32)

    for k in range(HM // ZW):
        pltpu.sync_copy(zbuf, acc_sh.at[b, pl.ds(half * HM + k * ZW, ZW)])
    plsc.subcore_barrier()

    acc_b = acc_sh.at[b]

    def start_inputs(g, slot):
        sv, vv, dv, _, sem, _ = slots[slot]
        base = ebase + g * CH
        pltpu.async_copy(src_hbm.at[pl.ds(base, CH)], sv, sem)
        pltpu.async_copy(val_hbm.at[pl.ds(base, CH)], vv, sem)
        pltpu.async_copy(dst_hbm.at[pl.ds(rbase + g * CR, CR)], dv, sem)

    def wait_inputs(slot):
        sv, vv, dv, _, sem, _ = slots[slot]
        pltpu.make_async_copy(src_hbm.at[pl.ds(0, CH)], sv, sem).wait()
        pltpu.make_async_copy(val_hbm.at[pl.ds(0, CH)], vv, sem).wait()
        pltpu.make_async_copy(dst_hbm.at[pl.ds(0, CR)], dv, sem).wait()

    def compute(slot):
        sv, vv, _, mv, _, _ = slots[slot]

        @plsc.parallel_loop(0, CH, 16, unroll=8)
        def _compute(i):
            idx = sv[pl.ds(i, 16)]
            val = vv[pl.ds(i, 16)]
            xg = plsc.load_gather(x_v, [idx])
            mv[pl.ds(i, 16)] = xg * val

    def issue_scatter(slot):
        _, _, dv, mv, _, sem = slots[slot]
        for r in range(CR):
            pltpu.async_copy(mv.at[pl.ds(r * 128, 128)], acc_b.at[dv.at[r]],
                             sem, add=True)

    def drain_scatter(slot):
        _, _, dv, mv, _, sem = slots[slot]
        for r in range(CR):
            pltpu.make_async_copy(mv.at[pl.ds(r * 128, 128)],
                                  acc_b.at[dv.at[r]], sem).wait()

    start_inputs(0, 0)
    start_inputs(1, 1)
    xcp.wait()

    @pl.loop(0, NCHUNK, step=NSLOT)
    def _main(g):
        for j in range(NSLOT):
            wait_inputs(j)
            compute(j)
            issue_scatter(j)
            nxt = (j + 2) % NSLOT
            if j < 2:
                @pl.when(g + j >= 2)
                def _():
                    drain_scatter(nxt)
            else:
                drain_scatter(nxt)

            @pl.when(g + j + 2 < NCHUNK)
            def _():
                start_inputs(g + j + 2, nxt)

    drain_scatter(2)
    drain_scatter(3)

    plsc.subcore_barrier()
    pltpu.sync_copy(acc_sh.at[b, pl.ds(half * HM, HM)],
                    out_hbm.at[c, b, pl.ds(half * HM, HM)])


def _combine_body(p_ref, b_ref, o_ref):
    o_ref[...] = p_ref[0] + p_ref[1] + b_ref[...]


def kernel(x, values, bias, indices):
    src = indices[0].astype(jnp.int32)
    dst = indices[1].astype(jnp.int32).reshape(E // 128, 128)
    xb = x.reshape(B, N)

    mesh = plsc.VectorSubcoreMesh(core_axis_name="c", subcore_axis_name="s")
    cp = pltpu.CompilerParams(use_tc_tiling_on_sc=False)
    if "needs_layout_passes" in pltpu.CompilerParams.__dataclass_fields__:
        cp = dataclasses.replace(cp, needs_layout_passes=False)
    buf_types = []
    for _ in range(NSLOT):
        buf_types += [
            pltpu.VMEM((CH,), jnp.int32),
            pltpu.VMEM((CH,), jnp.float32),
            pltpu.VMEM((CR, 128), jnp.int32),
            pltpu.VMEM((CH,), jnp.float32),
        ]
    sc = pl.kernel(
        _sc_kernel,
        out_type=jax.ShapeDtypeStruct((NC_CORES, B, M), jnp.float32),
        mesh=mesh,
        scratch_types=(
            [pltpu.VMEM((N,), jnp.float32)]
            + buf_types
            + [pltpu.VMEM((ZW,), jnp.float32),
               pltpu.VMEM_SHARED((B, M), jnp.float32)]
            + [pltpu.SemaphoreType.DMA] * (1 + 2 * NSLOT)
        ),
        compiler_params=cp,
    )
    partial = sc(xb, src, values, dst)

    bl = 2048
    out = pl.pallas_call(
        _combine_body,
        out_shape=jax.ShapeDtypeStruct((B, M), jnp.float32),
        grid=(M // bl,),
        in_specs=[
            pl.BlockSpec((NC_CORES, B, bl), lambda i: (0, 0, i)),
            pl.BlockSpec((1, bl), lambda i: (0, i)),
        ],
        out_specs=pl.BlockSpec((B, bl), lambda i: (0, i)),
    )(partial, bias.reshape(1, M))
    return out.reshape(B, M, 1)
